# Initial kernel scaffold; baseline (speedup 1.0000x reference)
import jax
import jax.numpy as jnp
from jax.experimental import pallas as pl


def kernel(x, noise):
    raise NotImplementedError("write your pallas kernel here")


# speedup vs baseline: 2.4007x; 2.4007x over previous
import functools

import jax
import jax.numpy as jnp
from jax import lax
from jax.experimental import pallas as pl
from jax.experimental.pallas import tpu as pltpu
from jax.experimental.pallas import tpu_sc as plsc

_MASK_RATIO = 0.75
_LANES = 16


@functools.lru_cache(maxsize=None)
def _build_mask_kernel(B, N, K):
    NW = 32
    rows_per_w = B // NW
    n_chunks = N // _LANES
    mesh = plsc.VectorSubcoreMesh(core_axis_name="c", subcore_axis_name="s")

    @functools.partial(
        pl.kernel,
        mesh=mesh,
        out_type=jax.ShapeDtypeStruct((B, N), jnp.float32),
        compiler_params=pltpu.CompilerParams(needs_layout_passes=False),
        scratch_types=[
            pltpu.VMEM((rows_per_w, N), jnp.int32),
            pltpu.VMEM((rows_per_w, N), jnp.float32),
        ],
    )
    def body(bits_hbm, out_hbm, bits_v, out_v):
        wid = lax.axis_index("s") * 2 + lax.axis_index("c")
        base = wid * rows_per_w
        pltpu.sync_copy(bits_hbm.at[pl.ds(base, rows_per_w)], bits_v)

        for r in range(rows_per_w):

            def bits_chunk(c):
                return bits_v[r, pl.ds(c * _LANES, _LANES)]

            def count_le(t):
                def cbody(c, acc):
                    return acc + (bits_chunk(c) <= t).astype(jnp.int32)
                acc = lax.fori_loop(0, n_chunks, cbody,
                                    jnp.zeros((_LANES,), jnp.int32))
                return jnp.sum(acc)

            def bs_body(i, carry):
                lo, hi = carry
                mid = lo + (hi - lo) // 2
                ge = count_le(mid) >= K
                return (jnp.where(ge, lo, mid + 1), jnp.where(ge, mid, hi))

            vstar, _ = lax.fori_loop(
                0, 31, bs_body,
                (jnp.int32(0), jnp.int32((1 << 31) - 1)))

            def lbody(c, acc):
                return acc + (bits_chunk(c) < vstar).astype(jnp.int32)

            c_less = jnp.sum(lax.fori_loop(0, n_chunks, lbody,
                                           jnp.zeros((_LANES,), jnp.int32)))
            rem = K - c_less

            def mbody(c, carry_eq):
                k = bits_chunk(c)
                eq = k == vstar
                eqi = eq.astype(jnp.int32)
                excl = jnp.cumsum(eqi) - eqi + carry_eq
                vis = (k < vstar) | (eq & (excl < rem))
                out_v[r, pl.ds(c * _LANES, _LANES)] = vis.astype(jnp.float32)
                return carry_eq + jnp.sum(eqi)

            lax.fori_loop(0, n_chunks, mbody, jnp.int32(0))

        pltpu.sync_copy(out_v, out_hbm.at[pl.ds(base, rows_per_w)])

    return body


def kernel(x, noise):
    B, N = x.shape[0], x.shape[1]
    num_visible = int(round(N * (1.0 - _MASK_RATIO)))
    num_visible = min(max(1, num_visible), N - 1)
    bits = lax.bitcast_convert_type(noise, jnp.int32)
    return _build_mask_kernel(B, N, num_visible)(bits)


# speedup vs baseline: 2.9133x; 1.2135x over previous
import functools

import jax
import jax.numpy as jnp
from jax import lax
from jax.experimental import pallas as pl
from jax.experimental.pallas import tpu as pltpu
from jax.experimental.pallas import tpu_sc as plsc

_MASK_RATIO = 0.75
_LANES = 16


@functools.lru_cache(maxsize=None)
def _build_mask_kernel(B, N, K):
    NW = 32
    rows_per_w = B // NW
    n_chunks = N // _LANES
    mesh = plsc.VectorSubcoreMesh(core_axis_name="c", subcore_axis_name="s")

    @functools.partial(
        pl.kernel,
        mesh=mesh,
        out_type=jax.ShapeDtypeStruct((B, N), jnp.float32),
        compiler_params=pltpu.CompilerParams(needs_layout_passes=False),
        scratch_types=[
            pltpu.VMEM((rows_per_w, N), jnp.int32),
            pltpu.VMEM((rows_per_w, N), jnp.float32),
        ],
    )
    def body(bits_hbm, out_hbm, bits_v, out_v):
        wid = lax.axis_index("s") * 2 + lax.axis_index("c")
        base = wid * rows_per_w
        pltpu.sync_copy(bits_hbm.at[pl.ds(base, rows_per_w)], bits_v)

        def chunk(r, c):
            return bits_v[r, pl.ds(c * _LANES, _LANES)]

        def count_le(ts):
            accs = [jnp.zeros((_LANES,), jnp.int32) for _ in ts]
            for c in range(n_chunks):
                for r, t in enumerate(ts):
                    accs[r] = accs[r] + (chunk(r, c) <= t).astype(jnp.int32)
            return [jnp.sum(a) for a in accs]

        def bs_body(i, carry):
            lohis = [carry[2 * r: 2 * r + 2] for r in range(rows_per_w)]
            mids = [lo + (hi - lo) // 2 for lo, hi in lohis]
            cnts = count_le(mids)
            out = []
            for (lo, hi), mid, cnt in zip(lohis, mids, cnts):
                ge = cnt >= K
                out += [jnp.where(ge, lo, mid + 1), jnp.where(ge, mid, hi)]
            return tuple(out)

        init = (jnp.int32(0), jnp.int32((1 << 30) - 1)) * rows_per_w
        res = lax.fori_loop(0, 30, bs_body, init)
        vstars = [res[2 * r] for r in range(rows_per_w)]

        accs = [jnp.zeros((_LANES,), jnp.int32) for _ in range(rows_per_w)]
        for c in range(n_chunks):
            for r in range(rows_per_w):
                accs[r] = accs[r] + (chunk(r, c) < vstars[r]).astype(jnp.int32)
        rems = [K - jnp.sum(a) for a in accs]

        carries = [jnp.int32(0) for _ in range(rows_per_w)]
        for c in range(n_chunks):
            for r in range(rows_per_w):
                k = chunk(r, c)
                eq = k == vstars[r]
                eqi = eq.astype(jnp.int32)
                excl = jnp.cumsum(eqi) - eqi + carries[r]
                vis = (k < vstars[r]) | (eq & (excl < rems[r]))
                out_v[r, pl.ds(c * _LANES, _LANES)] = vis.astype(jnp.float32)
                carries[r] = carries[r] + jnp.sum(eqi)

        pltpu.sync_copy(out_v, out_hbm.at[pl.ds(base, rows_per_w)])

    return body


def kernel(x, noise):
    B, N = x.shape[0], x.shape[1]
    num_visible = int(round(N * (1.0 - _MASK_RATIO)))
    num_visible = min(max(1, num_visible), N - 1)
    bits = lax.bitcast_convert_type(noise, jnp.int32)
    return _build_mask_kernel(B, N, num_visible)(bits)
